# Initial kernel scaffold; baseline (speedup 1.0000x reference)
import jax
import jax.numpy as jnp
from jax.experimental import pallas as pl


def kernel(input, target_g, gamma, bn_weight, bn_bias):
    raise NotImplementedError("write your pallas kernel here")


# speedup vs baseline: 192.3401x; 192.3401x over previous
import functools

import jax
import jax.numpy as jnp
from jax.experimental import pallas as pl

B, C, N = 4, 256, 2048
RB = 256
NB = N // RB
K = int(round(N * 0.8))
BISECT_ITERS = 32
EPS_NORM = 1e-12
EPS_BN = 1e-5
LEAKY = 0.01


def _fused_body(tn_ref, x_ref, o_ref, s1_ref, s2_ref):
    tn = tn_ref[0]
    x = x_ref[0]

    tnn = tn / jnp.maximum(jnp.sqrt(jnp.sum(tn * tn, axis=1, keepdims=True)),
                           EPS_NORM)
    xn = x / jnp.maximum(jnp.sqrt(jnp.sum(x * x, axis=1, keepdims=True)),
                         EPS_NORM)

    s = jax.lax.dot_general(tnn, xn, (((1,), (1,)), ((), ())),
                            preferred_element_type=jnp.float32)

    rmax = jnp.max(s, axis=1, keepdims=True)
    e = jnp.exp(s - rmax)
    denom = jnp.sum(e, axis=1, keepdims=True)

    lo = jnp.min(s, axis=1, keepdims=True)
    hi = rmax
    target = jnp.float32(K) - 0.5
    for _ in range(BISECT_ITERS):
        mid = 0.5 * (lo + hi)
        cnt = jnp.sum((s >= mid).astype(jnp.float32), axis=1, keepdims=True)
        ge = cnt > target
        lo = jnp.where(ge, mid, lo)
        hi = jnp.where(ge, hi, mid)

    a = jnp.where(s >= lo, e / denom, 0.0)

    o = jax.lax.dot_general(a, x, (((1,), (0,)), ((), ())),
                            preferred_element_type=jnp.float32)
    o = jnp.where(o >= 0, o, LEAKY * o)
    o_ref[0] = o
    s1_ref[0, 0, 0] = jnp.sum(o, axis=0)
    s2_ref[0, 0, 0] = jnp.sum(o * o, axis=0)


def _bn_body(o_ref, s1_ref, s2_ref, tg_ref, w_ref, b_ref, g_ref, out_ref):
    cnt = jnp.float32(B * N)
    tot = jnp.sum(s1_ref[...], axis=(0, 1, 2))
    totsq = jnp.sum(s2_ref[...], axis=(0, 1, 2))
    mean = tot / cnt
    var = totsq / cnt - mean * mean
    scale = w_ref[0] * jax.lax.rsqrt(var + EPS_BN)
    shift = b_ref[0] - mean * scale
    g = g_ref[0, 0]
    o = o_ref[0]
    y = (o * scale[None, :] + shift[None, :]) * g
    out_ref[0] = jnp.transpose(y) + tg_ref[0]


@jax.jit
def kernel(input, target_g, gamma, bn_weight, bn_bias):
    tnT = jnp.transpose(target_g, (0, 2, 1))
    xT = jnp.transpose(input, (0, 2, 1))

    o, s1, s2 = pl.pallas_call(
        _fused_body,
        grid=(B, NB),
        in_specs=[
            pl.BlockSpec((1, RB, C), lambda b, i: (b, i, 0)),
            pl.BlockSpec((1, N, C), lambda b, i: (b, 0, 0)),
        ],
        out_specs=[
            pl.BlockSpec((1, RB, C), lambda b, i: (b, i, 0)),
            pl.BlockSpec((1, 1, 1, C), lambda b, i: (b, i, 0, 0)),
            pl.BlockSpec((1, 1, 1, C), lambda b, i: (b, i, 0, 0)),
        ],
        out_shape=[
            jax.ShapeDtypeStruct((B, N, C), jnp.float32),
            jax.ShapeDtypeStruct((B, NB, 1, C), jnp.float32),
            jax.ShapeDtypeStruct((B, NB, 1, C), jnp.float32),
        ],
    )(tnT, xT)

    out = pl.pallas_call(
        _bn_body,
        grid=(B, NB),
        in_specs=[
            pl.BlockSpec((1, RB, C), lambda b, i: (b, i, 0)),
            pl.BlockSpec((B, NB, 1, C), lambda b, i: (0, 0, 0, 0)),
            pl.BlockSpec((B, NB, 1, C), lambda b, i: (0, 0, 0, 0)),
            pl.BlockSpec((1, C, RB), lambda b, i: (b, 0, i)),
            pl.BlockSpec((1, C), lambda b, i: (0, 0)),
            pl.BlockSpec((1, C), lambda b, i: (0, 0)),
            pl.BlockSpec((1, 1), lambda b, i: (0, 0)),
        ],
        out_specs=pl.BlockSpec((1, C, RB), lambda b, i: (b, 0, i)),
        out_shape=jax.ShapeDtypeStruct((B, C, N), jnp.float32),
    )(o, s1, s2, target_g, bn_weight.reshape(1, C), bn_bias.reshape(1, C),
      gamma.reshape(1, 1))
    return out


# speedup vs baseline: 255.7037x; 1.3294x over previous
import functools

import jax
import jax.numpy as jnp
from jax.experimental import pallas as pl

B, C, N = 4, 256, 2048
RB = 256
NB = N // RB
K = int(round(N * 0.8))
BISECT_ITERS = 20
EPS_NORM = 1e-12
EPS_BN = 1e-5
LEAKY = 0.01


def _fused_body(tn_ref, x_ref, o_ref, s1_ref, s2_ref):
    tn = tn_ref[0]
    x = x_ref[0]

    tnn = tn / jnp.maximum(jnp.sqrt(jnp.sum(tn * tn, axis=1, keepdims=True)),
                           EPS_NORM)
    xn = x / jnp.maximum(jnp.sqrt(jnp.sum(x * x, axis=1, keepdims=True)),
                         EPS_NORM)

    s = jax.lax.dot_general(tnn, xn, (((1,), (1,)), ((), ())),
                            preferred_element_type=jnp.float32)

    e = jnp.exp(s)
    denom = jnp.sum(e, axis=1, keepdims=True)

    lo = jnp.min(s, axis=1, keepdims=True)
    hi = jnp.max(s, axis=1, keepdims=True)
    target = jnp.float32(K) - 0.5
    for _ in range(BISECT_ITERS):
        mid = 0.5 * (lo + hi)
        cnt = jnp.sum((s >= mid).astype(jnp.float32), axis=1, keepdims=True)
        ge = cnt > target
        lo = jnp.where(ge, mid, lo)
        hi = jnp.where(ge, hi, mid)

    a = jnp.where(s >= lo, e, 0.0)

    o = jax.lax.dot_general(a, x, (((1,), (0,)), ((), ())),
                            preferred_element_type=jnp.float32)
    o = o * (1.0 / denom)
    o = jnp.where(o >= 0, o, LEAKY * o)
    o_ref[0] = o
    s1_ref[0, 0, 0] = jnp.sum(o, axis=0)
    s2_ref[0, 0, 0] = jnp.sum(o * o, axis=0)


def _bn_body(o_ref, s1_ref, s2_ref, tg_ref, w_ref, b_ref, g_ref, out_ref):
    cnt = jnp.float32(B * N)
    tot = jnp.sum(s1_ref[...], axis=(0, 1, 2))
    totsq = jnp.sum(s2_ref[...], axis=(0, 1, 2))
    mean = tot / cnt
    var = totsq / cnt - mean * mean
    scale = w_ref[0] * jax.lax.rsqrt(var + EPS_BN)
    shift = b_ref[0] - mean * scale
    g = g_ref[0, 0]
    o = o_ref[0]
    y = (o * scale[None, :] + shift[None, :]) * g
    out_ref[0] = jnp.transpose(y) + tg_ref[0]


@jax.jit
def kernel(input, target_g, gamma, bn_weight, bn_bias):
    tnT = jnp.transpose(target_g, (0, 2, 1))
    xT = jnp.transpose(input, (0, 2, 1))

    o, s1, s2 = pl.pallas_call(
        _fused_body,
        grid=(B, NB),
        in_specs=[
            pl.BlockSpec((1, RB, C), lambda b, i: (b, i, 0)),
            pl.BlockSpec((1, N, C), lambda b, i: (b, 0, 0)),
        ],
        out_specs=[
            pl.BlockSpec((1, RB, C), lambda b, i: (b, i, 0)),
            pl.BlockSpec((1, 1, 1, C), lambda b, i: (b, i, 0, 0)),
            pl.BlockSpec((1, 1, 1, C), lambda b, i: (b, i, 0, 0)),
        ],
        out_shape=[
            jax.ShapeDtypeStruct((B, N, C), jnp.float32),
            jax.ShapeDtypeStruct((B, NB, 1, C), jnp.float32),
            jax.ShapeDtypeStruct((B, NB, 1, C), jnp.float32),
        ],
    )(tnT, xT)

    out = pl.pallas_call(
        _bn_body,
        grid=(B, NB),
        in_specs=[
            pl.BlockSpec((1, RB, C), lambda b, i: (b, i, 0)),
            pl.BlockSpec((B, NB, 1, C), lambda b, i: (0, 0, 0, 0)),
            pl.BlockSpec((B, NB, 1, C), lambda b, i: (0, 0, 0, 0)),
            pl.BlockSpec((1, C, RB), lambda b, i: (b, 0, i)),
            pl.BlockSpec((1, C), lambda b, i: (0, 0)),
            pl.BlockSpec((1, C), lambda b, i: (0, 0)),
            pl.BlockSpec((1, 1), lambda b, i: (0, 0)),
        ],
        out_specs=pl.BlockSpec((1, C, RB), lambda b, i: (b, 0, i)),
        out_shape=jax.ShapeDtypeStruct((B, C, N), jnp.float32),
    )(o, s1, s2, target_g, bn_weight.reshape(1, C), bn_bias.reshape(1, C),
      gamma.reshape(1, 1))
    return out


# speedup vs baseline: 267.3592x; 1.0456x over previous
import functools

import jax
import jax.numpy as jnp
from jax.experimental import pallas as pl

B, C, N = 4, 256, 2048
RB = 256
NB = N // RB
K = int(round(N * 0.8))
BISECT_ITERS = 18
EPS_NORM = 1e-12
EPS_BN = 1e-5
LEAKY = 0.01


def _norm_body(tn_ref, x_ref, tno_ref, xno_ref):
    tn = tn_ref[0]
    x = x_ref[0]
    tno_ref[0] = tn / jnp.maximum(
        jnp.sqrt(jnp.sum(tn * tn, axis=1, keepdims=True)), EPS_NORM)
    xno_ref[0] = x / jnp.maximum(
        jnp.sqrt(jnp.sum(x * x, axis=1, keepdims=True)), EPS_NORM)


def _fused_body(tnn_ref, xn_ref, x_ref, o_ref, s1_ref, s2_ref):
    tnn = tnn_ref[0]
    xn = xn_ref[0]
    x = x_ref[0]

    s = jax.lax.dot_general(tnn, xn, (((1,), (1,)), ((), ())),
                            preferred_element_type=jnp.float32)

    e = jnp.exp(s)
    denom = jax.lax.dot_general(e, jnp.ones((N, 1), jnp.float32),
                                (((1,), (0,)), ((), ())),
                                preferred_element_type=jnp.float32)

    lo = jnp.min(s, axis=1, keepdims=True)
    hi = jnp.max(s, axis=1, keepdims=True)
    target = jnp.float32(K) - 0.5
    for _ in range(BISECT_ITERS):
        mid = 0.5 * (lo + hi)
        cnt = jnp.sum((s >= mid).astype(jnp.float32), axis=1, keepdims=True)
        ge = cnt > target
        lo = jnp.where(ge, mid, lo)
        hi = jnp.where(ge, hi, mid)

    a = jnp.where(s >= lo, e, 0.0)

    o = jax.lax.dot_general(a, x, (((1,), (0,)), ((), ())),
                            preferred_element_type=jnp.float32)
    o = o * (1.0 / denom)
    o = jnp.where(o >= 0, o, LEAKY * o)
    o_ref[0] = o
    s1_ref[0, 0, 0] = jnp.sum(o, axis=0)
    s2_ref[0, 0, 0] = jnp.sum(o * o, axis=0)


def _bn_body(o_ref, s1_ref, s2_ref, tg_ref, w_ref, b_ref, g_ref, out_ref):
    cnt = jnp.float32(B * N)
    tot = jnp.sum(s1_ref[...], axis=(0, 1, 2))
    totsq = jnp.sum(s2_ref[...], axis=(0, 1, 2))
    mean = tot / cnt
    var = totsq / cnt - mean * mean
    scale = w_ref[0] * jax.lax.rsqrt(var + EPS_BN)
    shift = b_ref[0] - mean * scale
    g = g_ref[0, 0]
    o = o_ref[0]
    y = (o * scale[None, :] + shift[None, :]) * g
    out_ref[0] = jnp.transpose(y) + tg_ref[0]


@jax.jit
def kernel(input, target_g, gamma, bn_weight, bn_bias):
    tnT = jnp.transpose(target_g, (0, 2, 1))
    xT = jnp.transpose(input, (0, 2, 1))

    tnn, xn = pl.pallas_call(
        _norm_body,
        grid=(B,),
        in_specs=[
            pl.BlockSpec((1, N, C), lambda b: (b, 0, 0)),
            pl.BlockSpec((1, N, C), lambda b: (b, 0, 0)),
        ],
        out_specs=[
            pl.BlockSpec((1, N, C), lambda b: (b, 0, 0)),
            pl.BlockSpec((1, N, C), lambda b: (b, 0, 0)),
        ],
        out_shape=[
            jax.ShapeDtypeStruct((B, N, C), jnp.float32),
            jax.ShapeDtypeStruct((B, N, C), jnp.float32),
        ],
    )(tnT, xT)

    o, s1, s2 = pl.pallas_call(
        _fused_body,
        grid=(B, NB),
        in_specs=[
            pl.BlockSpec((1, RB, C), lambda b, i: (b, i, 0)),
            pl.BlockSpec((1, N, C), lambda b, i: (b, 0, 0)),
            pl.BlockSpec((1, N, C), lambda b, i: (b, 0, 0)),
        ],
        out_specs=[
            pl.BlockSpec((1, RB, C), lambda b, i: (b, i, 0)),
            pl.BlockSpec((1, 1, 1, C), lambda b, i: (b, i, 0, 0)),
            pl.BlockSpec((1, 1, 1, C), lambda b, i: (b, i, 0, 0)),
        ],
        out_shape=[
            jax.ShapeDtypeStruct((B, N, C), jnp.float32),
            jax.ShapeDtypeStruct((B, NB, 1, C), jnp.float32),
            jax.ShapeDtypeStruct((B, NB, 1, C), jnp.float32),
        ],
    )(tnn, xn, xT)

    out = pl.pallas_call(
        _bn_body,
        grid=(B, NB),
        in_specs=[
            pl.BlockSpec((1, RB, C), lambda b, i: (b, i, 0)),
            pl.BlockSpec((B, NB, 1, C), lambda b, i: (0, 0, 0, 0)),
            pl.BlockSpec((B, NB, 1, C), lambda b, i: (0, 0, 0, 0)),
            pl.BlockSpec((1, C, RB), lambda b, i: (b, 0, i)),
            pl.BlockSpec((1, C), lambda b, i: (0, 0)),
            pl.BlockSpec((1, C), lambda b, i: (0, 0)),
            pl.BlockSpec((1, 1), lambda b, i: (0, 0)),
        ],
        out_specs=pl.BlockSpec((1, C, RB), lambda b, i: (b, 0, i)),
        out_shape=jax.ShapeDtypeStruct((B, C, N), jnp.float32),
    )(o, s1, s2, target_g, bn_weight.reshape(1, C), bn_bias.reshape(1, C),
      gamma.reshape(1, 1))
    return out


# speedup vs baseline: 314.2246x; 1.1753x over previous
import functools

import jax
import jax.numpy as jnp
from jax.experimental import pallas as pl

B, C, N = 4, 256, 2048
RB = 256
NB = N // RB
K = int(round(N * 0.8))
BISECT_ITERS = 18
EPS_NORM = 1e-12
EPS_BN = 1e-5
LEAKY = 0.01


def _norm_body(tg_ref, x_ref, tgo_ref, xo_ref):
    tg = tg_ref[0]
    x = x_ref[0]
    tgo_ref[0] = tg / jnp.maximum(
        jnp.sqrt(jnp.sum(tg * tg, axis=0, keepdims=True)), EPS_NORM)
    xo_ref[0] = x / jnp.maximum(
        jnp.sqrt(jnp.sum(x * x, axis=0, keepdims=True)), EPS_NORM)


def _fused_body(tnn_ref, xn_ref, x_ref, o_ref, s1_ref, s2_ref):
    tnn = tnn_ref[0]
    xn = xn_ref[0]
    x = x_ref[0]

    s = jax.lax.dot_general(tnn, xn, (((0,), (0,)), ((), ())),
                            preferred_element_type=jnp.float32)

    e = jnp.exp(s)
    denom = jax.lax.dot_general(e, jnp.ones((N, 1), jnp.float32),
                                (((1,), (0,)), ((), ())),
                                preferred_element_type=jnp.float32)

    lo = jnp.min(s, axis=1, keepdims=True)
    hi = jnp.max(s, axis=1, keepdims=True)
    target = jnp.float32(K) - 0.5
    for _ in range(BISECT_ITERS):
        mid = 0.5 * (lo + hi)
        cnt = jnp.sum((s >= mid).astype(jnp.float32), axis=1, keepdims=True)
        ge = cnt > target
        lo = jnp.where(ge, mid, lo)
        hi = jnp.where(ge, hi, mid)

    a = jnp.where(s >= lo, e, 0.0)

    o = jax.lax.dot_general(a, x, (((1,), (1,)), ((), ())),
                            preferred_element_type=jnp.float32)
    o = o * (1.0 / denom)
    o = jnp.where(o >= 0, o, LEAKY * o)
    o_ref[0] = o
    s1_ref[0, 0, 0] = jnp.sum(o, axis=0)
    s2_ref[0, 0, 0] = jnp.sum(o * o, axis=0)


def _bn_body(o_ref, s1_ref, s2_ref, tg_ref, w_ref, b_ref, g_ref, out_ref):
    cnt = jnp.float32(B * N)
    tot = jnp.sum(s1_ref[...], axis=(0, 1, 2))
    totsq = jnp.sum(s2_ref[...], axis=(0, 1, 2))
    mean = tot / cnt
    var = totsq / cnt - mean * mean
    scale = w_ref[0] * jax.lax.rsqrt(var + EPS_BN)
    shift = b_ref[0] - mean * scale
    g = g_ref[0, 0]
    o = o_ref[0]
    y = (o * scale[None, :] + shift[None, :]) * g
    out_ref[0] = jnp.transpose(y) + tg_ref[0]


@jax.jit
def kernel(input, target_g, gamma, bn_weight, bn_bias):
    tnn, xn = pl.pallas_call(
        _norm_body,
        grid=(B,),
        in_specs=[
            pl.BlockSpec((1, C, N), lambda b: (b, 0, 0)),
            pl.BlockSpec((1, C, N), lambda b: (b, 0, 0)),
        ],
        out_specs=[
            pl.BlockSpec((1, C, N), lambda b: (b, 0, 0)),
            pl.BlockSpec((1, C, N), lambda b: (b, 0, 0)),
        ],
        out_shape=[
            jax.ShapeDtypeStruct((B, C, N), jnp.float32),
            jax.ShapeDtypeStruct((B, C, N), jnp.float32),
        ],
    )(target_g, input)

    o, s1, s2 = pl.pallas_call(
        _fused_body,
        grid=(B, NB),
        in_specs=[
            pl.BlockSpec((1, C, RB), lambda b, i: (b, 0, i)),
            pl.BlockSpec((1, C, N), lambda b, i: (b, 0, 0)),
            pl.BlockSpec((1, C, N), lambda b, i: (b, 0, 0)),
        ],
        out_specs=[
            pl.BlockSpec((1, RB, C), lambda b, i: (b, i, 0)),
            pl.BlockSpec((1, 1, 1, C), lambda b, i: (b, i, 0, 0)),
            pl.BlockSpec((1, 1, 1, C), lambda b, i: (b, i, 0, 0)),
        ],
        out_shape=[
            jax.ShapeDtypeStruct((B, N, C), jnp.float32),
            jax.ShapeDtypeStruct((B, NB, 1, C), jnp.float32),
            jax.ShapeDtypeStruct((B, NB, 1, C), jnp.float32),
        ],
    )(tnn, xn, input)

    out = pl.pallas_call(
        _bn_body,
        grid=(B, NB),
        in_specs=[
            pl.BlockSpec((1, RB, C), lambda b, i: (b, i, 0)),
            pl.BlockSpec((B, NB, 1, C), lambda b, i: (0, 0, 0, 0)),
            pl.BlockSpec((B, NB, 1, C), lambda b, i: (0, 0, 0, 0)),
            pl.BlockSpec((1, C, RB), lambda b, i: (b, 0, i)),
            pl.BlockSpec((1, C), lambda b, i: (0, 0)),
            pl.BlockSpec((1, C), lambda b, i: (0, 0)),
            pl.BlockSpec((1, 1), lambda b, i: (0, 0)),
        ],
        out_specs=pl.BlockSpec((1, C, RB), lambda b, i: (b, 0, i)),
        out_shape=jax.ShapeDtypeStruct((B, C, N), jnp.float32),
    )(o, s1, s2, target_g, bn_weight.reshape(1, C), bn_bias.reshape(1, C),
      gamma.reshape(1, 1))
    return out
